# baseline (speedup 1.0000x reference)
import functools

import jax
import jax.numpy as jnp
from jax import lax
from jax.experimental import pallas as pl
from jax.experimental.pallas import tpu as pltpu
from jax.experimental.pallas import tpu_sc as plsc

H = 128
W = 128
N_PIXELS = H * W
N_ENTRIES = 524288
N_GENES = 20000
DIMS = 64
FFN_DIMS = 192
N_LAYERS = 3
N_HEADS = 4
HEAD = DIMS // N_HEADS
COUNTS_BIN = (1, 2, 4, 8, 16, 32, 64)
EPS = float(jnp.finfo('float32').eps)

NC = 2
NS = 16
NW = NC * NS
CHUNK = 128
YCOLS = 80
BE = 2048
BP = 2048

_PREC = jax.lax.Precision.HIGHEST


@functools.lru_cache(maxsize=None)
def _make_gather(n_idx, dims):
    per_w = n_idx // NW
    n_ch = per_w // CHUNK
    mesh = plsc.VectorSubcoreMesh(core_axis_name="c", subcore_axis_name="s",
                                  num_cores=NC)

    def body(table_hbm, idx_hbm, out_hbm, idx_v, rows_v, sem):
        wid = lax.axis_index("s") * NC + lax.axis_index("c")
        base = wid * per_w

        def step(i, carry):
            off = base + i * CHUNK
            pltpu.sync_copy(idx_hbm.at[pl.ds(off, CHUNK)], idx_v)
            pltpu.async_copy(table_hbm.at[idx_v], rows_v, sem).wait()
            pltpu.sync_copy(rows_v, out_hbm.at[pl.ds(off, CHUNK)])
            return carry

        lax.fori_loop(0, n_ch, step, 0)

    return pl.kernel(
        body,
        out_type=jax.ShapeDtypeStruct((n_idx, dims), jnp.float32),
        mesh=mesh,
        compiler_params=pltpu.CompilerParams(use_tc_tiling_on_sc=False),
        scratch_types=[
            pltpu.VMEM((CHUNK,), jnp.int32),
            pltpu.VMEM((CHUNK, dims), jnp.float32),
            pltpu.SemaphoreType.DMA,
        ],
    )


@functools.lru_cache(maxsize=None)
def _make_seg_reduce():
    per_w = N_ENTRIES // NW
    n_ch = per_w // CHUNK
    rows_per_s = N_PIXELS // NS
    mesh = plsc.VectorSubcoreMesh(core_axis_name="c", subcore_axis_name="s",
                                  num_cores=NC)

    def body(y_hbm, seg_hbm, z_hbm, out_hbm, y_v, seg_v, acc):
        cid = lax.axis_index("c")
        sid = lax.axis_index("s")
        wid = sid * NC + cid
        pltpu.sync_copy(z_hbm, acc.at[pl.ds(sid * rows_per_s, rows_per_s)])
        plsc.subcore_barrier()
        base = wid * per_w

        def step(i, carry):
            off = base + i * CHUNK
            pltpu.sync_copy(y_hbm.at[pl.ds(off, CHUNK)], y_v)
            pltpu.sync_copy(seg_hbm.at[pl.ds(off, CHUNK)], seg_v)
            pltpu.sync_copy(y_v, acc.at[seg_v], add=True)
            return carry

        lax.fori_loop(0, n_ch, step, 0)
        plsc.subcore_barrier()
        pltpu.sync_copy(acc.at[pl.ds(sid * rows_per_s, rows_per_s)],
                        out_hbm.at[cid, pl.ds(sid * rows_per_s, rows_per_s)])

    return pl.kernel(
        body,
        out_type=jax.ShapeDtypeStruct((NC, N_PIXELS, YCOLS), jnp.float32),
        mesh=mesh,
        compiler_params=pltpu.CompilerParams(use_tc_tiling_on_sc=False),
        scratch_types=[
            pltpu.VMEM((CHUNK, YCOLS), jnp.float32),
            pltpu.VMEM((CHUNK,), jnp.int32),
            pltpu.VMEM_SHARED((N_PIXELS, YCOLS), jnp.float32),
        ],
    )


def _block_diag_expand(rows, cols):
    r = lax.broadcasted_iota(jnp.int32, (rows, cols), 0)
    c = lax.broadcasted_iota(jnp.int32, (rows, cols), 1)
    if rows >= cols:
        return (r // (rows // cols) == c).astype(jnp.float32)
    return (c // (cols // rows) == r).astype(jnp.float32)


def _onehot_cnts(cnts_blk):
    c = cnts_blk
    dig = jnp.zeros_like(c)
    for b in COUNTS_BIN:
        dig = dig + (c >= b).astype(jnp.int32)
    bins = lax.broadcasted_iota(jnp.int32, (1, len(COUNTS_BIN) + 1), 1)
    return (dig == bins).astype(jnp.float32)


def _folded_weights(ce, cw, cb, pw, pb):
    top = jnp.dot(cw, pw, precision=_PREC)
    bot = jnp.dot(jnp.dot(ce, cw, precision=_PREC), pw, precision=_PREC)
    wfull = jnp.concatenate([top, bot], axis=0)
    bfull = jnp.dot(cb[None, :], pw, precision=_PREC)[0] + pb
    return wfull, bfull


def _entry_l1_kernel(gemb_ref, cnts_ref, ce_ref, cw_ref, cb_ref,
                     vw_ref, vb_ref, y_ref):
    g = gemb_ref[...]
    oh = _onehot_cnts(cnts_ref[...])
    a = jnp.concatenate([g, oh], axis=1)
    wv, bv = _folded_weights(ce_ref[...], cw_ref[...], cb_ref[...],
                             vw_ref[...], vb_ref[...])
    v = jnp.dot(a, wv, precision=_PREC) + bv
    ones = jnp.ones((v.shape[0], N_HEADS), jnp.float32)
    pad = jnp.zeros((v.shape[0], YCOLS - DIMS - N_HEADS), jnp.float32)
    y_ref[...] = jnp.concatenate([v, ones, pad], axis=1)


def _entry_kernel(gemb_ref, cnts_ref, qexp_ref, ce_ref, cw_ref, cb_ref,
                  kw_ref, kb_ref, vw_ref, vb_ref, y_ref):
    g = gemb_ref[...]
    oh = _onehot_cnts(cnts_ref[...])
    a = jnp.concatenate([g, oh], axis=1)
    wk, bk = _folded_weights(ce_ref[...], cw_ref[...], cb_ref[...],
                             kw_ref[...], kb_ref[...])
    wv, bv = _folded_weights(ce_ref[...], cw_ref[...], cb_ref[...],
                             vw_ref[...], vb_ref[...])
    k = jnp.dot(a, wk, precision=_PREC) + bk
    v = jnp.dot(a, wv, precision=_PREC) + bv
    q = qexp_ref[...]
    hs = _block_diag_expand(DIMS, N_HEADS)
    he = _block_diag_expand(N_HEADS, DIMS)
    wlog = jnp.dot(q * k, hs, precision=_PREC)
    e = jnp.exp(wlog)
    ev = v * jnp.dot(e, he, precision=_PREC)
    pad = jnp.zeros((e.shape[0], YCOLS - DIMS - N_HEADS), jnp.float32)
    y_ref[...] = jnp.concatenate([ev, e, pad], axis=1)


def _ln_block(x, scale, bias):
    m = x.mean(axis=-1, keepdims=True)
    v = jnp.mean(jnp.square(x - m), axis=-1, keepdims=True)
    return (x - m) / jnp.sqrt(v + 1e-6) * scale + bias


def _pixel_kernel(acc_ref, tok_ref, ow_ref, ob_ref, ls_ref, lb_ref,
                  w1_ref, b1_ref, w2_ref, b2_ref, fs_ref, fb_ref,
                  qw_ref, qb_ref, x_ref, q_ref):
    s = acc_ref[0] + acc_ref[1]
    numer = s[:, :DIMS]
    norm = s[:, DIMS:DIMS + N_HEADS]
    he = _block_diag_expand(N_HEADS, DIMS)
    dinv = 1.0 / (norm + EPS)
    r = numer * jnp.dot(dinv, he, precision=_PREC)
    res = jnp.dot(r, ow_ref[...], precision=_PREC) + ob_ref[...] + tok_ref[...]
    h = _ln_block(res, ls_ref[...], lb_ref[...])
    f = jnp.dot(jax.nn.gelu(jnp.dot(h, w1_ref[...], precision=_PREC)
                            + b1_ref[...]),
                w2_ref[...], precision=_PREC) + b2_ref[...]
    x = _ln_block(h + f, fs_ref[...], fb_ref[...])
    x_ref[...] = x
    q_ref[...] = (jnp.dot(x, qw_ref[...], precision=_PREC)
                  + qb_ref[...]) * (1.0 / jnp.sqrt(jnp.float32(HEAD)))


def _mat_spec(shape):
    nd = len(shape)
    return pl.BlockSpec(shape, lambda i: (0,) * nd)


def _entry_l1_call(gemb, cnts2, ce, cw, cb, vw, vb):
    grid = (N_ENTRIES // BE,)
    return pl.pallas_call(
        _entry_l1_kernel,
        grid=grid,
        in_specs=[
            pl.BlockSpec((BE, DIMS), lambda i: (i, 0)),
            pl.BlockSpec((BE, 1), lambda i: (i, 0)),
            _mat_spec((len(COUNTS_BIN) + 1, DIMS)),
            _mat_spec((DIMS, DIMS)),
            _mat_spec((DIMS,)),
            _mat_spec((DIMS, DIMS)),
            _mat_spec((DIMS,)),
        ],
        out_specs=pl.BlockSpec((BE, YCOLS), lambda i: (i, 0)),
        out_shape=jax.ShapeDtypeStruct((N_ENTRIES, YCOLS), jnp.float32),
    )(gemb, cnts2, ce, cw, cb, vw, vb)


def _entry_call(gemb, cnts2, qexp, ce, cw, cb, kw, kb, vw, vb):
    grid = (N_ENTRIES // BE,)
    return pl.pallas_call(
        _entry_kernel,
        grid=grid,
        in_specs=[
            pl.BlockSpec((BE, DIMS), lambda i: (i, 0)),
            pl.BlockSpec((BE, 1), lambda i: (i, 0)),
            pl.BlockSpec((BE, DIMS), lambda i: (i, 0)),
            _mat_spec((len(COUNTS_BIN) + 1, DIMS)),
            _mat_spec((DIMS, DIMS)),
            _mat_spec((DIMS,)),
            _mat_spec((DIMS, DIMS)),
            _mat_spec((DIMS,)),
            _mat_spec((DIMS, DIMS)),
            _mat_spec((DIMS,)),
        ],
        out_specs=pl.BlockSpec((BE, YCOLS), lambda i: (i, 0)),
        out_shape=jax.ShapeDtypeStruct((N_ENTRIES, YCOLS), jnp.float32),
    )(gemb, cnts2, qexp, ce, cw, cb, kw, kb, vw, vb)


def _pixel_call(acc, tok, p, pn):
    grid = (N_PIXELS // BP,)
    return pl.pallas_call(
        _pixel_kernel,
        grid=grid,
        in_specs=[
            pl.BlockSpec((NC, BP, YCOLS), lambda i: (0, i, 0)),
            pl.BlockSpec((BP, DIMS), lambda i: (i, 0)),
            _mat_spec((DIMS, DIMS)),
            _mat_spec((DIMS,)),
            _mat_spec((DIMS,)),
            _mat_spec((DIMS,)),
            _mat_spec((DIMS, FFN_DIMS)),
            _mat_spec((FFN_DIMS,)),
            _mat_spec((FFN_DIMS, DIMS)),
            _mat_spec((DIMS,)),
            _mat_spec((DIMS,)),
            _mat_spec((DIMS,)),
            _mat_spec((DIMS, DIMS)),
            _mat_spec((DIMS,)),
        ],
        out_specs=[
            pl.BlockSpec((BP, DIMS), lambda i: (i, 0)),
            pl.BlockSpec((BP, DIMS), lambda i: (i, 0)),
        ],
        out_shape=[
            jax.ShapeDtypeStruct((N_PIXELS, DIMS), jnp.float32),
            jax.ShapeDtypeStruct((N_PIXELS, DIMS), jnp.float32),
        ],
    )(acc, tok, p['o_W'], p['o_b'], p['att_ln_s'], p['att_ln_b'],
      p['ffn_W1'], p['ffn_b1'], p['ffn_W2'], p['ffn_b2'],
      p['ffn_ln_s'], p['ffn_ln_b'], pn['q_W'], pn['q_b'])


def _gene_gather(table, idx):
    return _make_gather(N_ENTRIES, DIMS)(table, idx)


def _q_gather(table, idx):
    return _make_gather(N_ENTRIES, DIMS)(table, idx)


def _seg_reduce(y, segms, zrows):
    return _make_seg_reduce()(y, segms, zrows)


def kernel(indices, indptr, cnts, params):
    segms = jnp.searchsorted(indptr[1:], jnp.arange(N_ENTRIES, dtype=jnp.int32),
                             side='right').astype(jnp.int32)
    cnts2 = cnts.reshape(N_ENTRIES, 1)
    zrows = jnp.zeros((N_PIXELS // NS, YCOLS), jnp.float32)
    tok0 = jnp.zeros((N_PIXELS, DIMS), jnp.float32)

    gemb = _gene_gather(params['gene_embed'], indices)

    ce, cw, cb = params['cnt_embed'], params['ctx_W'], params['ctx_b']
    layers = params['layers']

    y = _entry_l1_call(gemb, cnts2, ce, cw, cb,
                       layers[0]['v_W'], layers[0]['v_b'])
    acc = _seg_reduce(y, segms, zrows)
    x, q = _pixel_call(acc, tok0, layers[0], layers[1])

    for li in (1, 2):
        p = layers[li]
        qexp = _q_gather(q, segms)
        y = _entry_call(gemb, cnts2, qexp, ce, cw, cb,
                        p['k_W'], p['k_b'], p['v_W'], p['v_b'])
        acc = _seg_reduce(y, segms, zrows)
        pn = layers[li + 1] if li + 1 < N_LAYERS else p
        x, q = _pixel_call(acc, x, p, pn)

    return x.reshape(H, W, DIMS)


# speedup vs baseline: 10.9571x; 10.9571x over previous
import functools

import jax
import jax.numpy as jnp
from jax import lax
from jax.experimental import pallas as pl
from jax.experimental.pallas import tpu as pltpu
from jax.experimental.pallas import tpu_sc as plsc

H = 128
W = 128
N_PIXELS = H * W
N_ENTRIES = 524288
N_GENES = 20000
DIMS = 64
FFN_DIMS = 192
N_LAYERS = 3
N_HEADS = 4
HEAD = DIMS // N_HEADS
COUNTS_BIN = (1, 2, 4, 8, 16, 32, 64)
EPS = float(jnp.finfo('float32').eps)

NC = 2
NS = 16
NW = NC * NS
CHUNK = 128
YCOLS = 80
BE = 2048
BP = 2048

_PREC = jax.lax.Precision.HIGHEST


@functools.lru_cache(maxsize=None)
def _make_gather(n_idx, dims):
    per_w = n_idx // NW
    n_ch = per_w // CHUNK
    mesh = plsc.VectorSubcoreMesh(core_axis_name="c", subcore_axis_name="s",
                                  num_cores=NC)

    def body(table_hbm, idx_hbm, out_hbm, idx_v, rows_v, sem):
        wid = lax.axis_index("s") * NC + lax.axis_index("c")
        base = wid * per_w

        def step(i, carry):
            off = base + i * CHUNK
            pltpu.sync_copy(idx_hbm.at[pl.ds(off, CHUNK)], idx_v)
            pltpu.async_copy(table_hbm.at[idx_v], rows_v, sem).wait()
            pltpu.sync_copy(rows_v, out_hbm.at[pl.ds(off, CHUNK)])
            return carry

        lax.fori_loop(0, n_ch, step, 0)

    return pl.kernel(
        body,
        out_type=jax.ShapeDtypeStruct((n_idx, dims), jnp.float32),
        mesh=mesh,
        compiler_params=pltpu.CompilerParams(use_tc_tiling_on_sc=False),
        scratch_types=[
            pltpu.VMEM((CHUNK,), jnp.int32),
            pltpu.VMEM((CHUNK, dims), jnp.float32),
            pltpu.SemaphoreType.DMA,
        ],
    )


@functools.lru_cache(maxsize=None)
def _make_seg_reduce():
    per_w = N_ENTRIES // NW
    n_ch = per_w // CHUNK
    rows_per_s = N_PIXELS // NS
    mesh = plsc.VectorSubcoreMesh(core_axis_name="c", subcore_axis_name="s",
                                  num_cores=NC)

    def body(y_hbm, seg_hbm, z_hbm, out_hbm, y_v, seg_v, acc):
        cid = lax.axis_index("c")
        sid = lax.axis_index("s")
        wid = sid * NC + cid
        pltpu.sync_copy(z_hbm, acc.at[pl.ds(sid * rows_per_s, rows_per_s)])
        plsc.subcore_barrier()
        base = wid * per_w

        def step(i, carry):
            off = base + i * CHUNK
            pltpu.sync_copy(y_hbm.at[pl.ds(off, CHUNK)], y_v)
            pltpu.sync_copy(seg_hbm.at[pl.ds(off, CHUNK)], seg_v)
            pltpu.sync_copy(y_v, acc.at[seg_v], add=True)
            return carry

        lax.fori_loop(0, n_ch, step, 0)
        plsc.subcore_barrier()
        pltpu.sync_copy(acc.at[pl.ds(sid * rows_per_s, rows_per_s)],
                        out_hbm.at[cid, pl.ds(sid * rows_per_s, rows_per_s)])

    return pl.kernel(
        body,
        out_type=jax.ShapeDtypeStruct((NC, N_PIXELS, YCOLS), jnp.float32),
        mesh=mesh,
        compiler_params=pltpu.CompilerParams(use_tc_tiling_on_sc=False),
        scratch_types=[
            pltpu.VMEM((CHUNK, YCOLS), jnp.float32),
            pltpu.VMEM((CHUNK,), jnp.int32),
            pltpu.VMEM_SHARED((N_PIXELS, YCOLS), jnp.float32),
        ],
    )


def _block_diag_expand(rows, cols):
    r = lax.broadcasted_iota(jnp.int32, (rows, cols), 0)
    c = lax.broadcasted_iota(jnp.int32, (rows, cols), 1)
    if rows >= cols:
        return (r // (rows // cols) == c).astype(jnp.float32)
    return (c // (cols // rows) == r).astype(jnp.float32)


def _onehot_cnts(cnts_blk):
    c = cnts_blk
    dig = jnp.zeros_like(c)
    for b in COUNTS_BIN:
        dig = dig + (c >= b).astype(jnp.int32)
    bins = lax.broadcasted_iota(jnp.int32, (1, len(COUNTS_BIN) + 1), 1)
    return (dig == bins).astype(jnp.float32)


def _folded_weights(ce, cw, cb, pw, pb):
    top = jnp.dot(cw, pw, precision=_PREC)
    bot = jnp.dot(jnp.dot(ce, cw, precision=_PREC), pw, precision=_PREC)
    wfull = jnp.concatenate([top, bot], axis=0)
    bfull = jnp.dot(cb[None, :], pw, precision=_PREC)[0] + pb
    return wfull, bfull


def _entry_l1_kernel(gemb_ref, cnts_ref, ce_ref, cw_ref, cb_ref,
                     vw_ref, vb_ref, y_ref):
    g = gemb_ref[...]
    oh = _onehot_cnts(cnts_ref[...])
    a = jnp.concatenate([g, oh], axis=1)
    wv, bv = _folded_weights(ce_ref[...], cw_ref[...], cb_ref[...],
                             vw_ref[...], vb_ref[...])
    v = jnp.dot(a, wv, precision=_PREC) + bv
    ones = jnp.ones((v.shape[0], N_HEADS), jnp.float32)
    pad = jnp.zeros((v.shape[0], YCOLS - DIMS - N_HEADS), jnp.float32)
    y_ref[...] = jnp.concatenate([v, ones, pad], axis=1)


def _entry_kernel(gemb_ref, cnts_ref, qexp_ref, ce_ref, cw_ref, cb_ref,
                  kw_ref, kb_ref, vw_ref, vb_ref, y_ref):
    g = gemb_ref[...]
    oh = _onehot_cnts(cnts_ref[...])
    a = jnp.concatenate([g, oh], axis=1)
    wk, bk = _folded_weights(ce_ref[...], cw_ref[...], cb_ref[...],
                             kw_ref[...], kb_ref[...])
    wv, bv = _folded_weights(ce_ref[...], cw_ref[...], cb_ref[...],
                             vw_ref[...], vb_ref[...])
    k = jnp.dot(a, wk, precision=_PREC) + bk
    v = jnp.dot(a, wv, precision=_PREC) + bv
    q = qexp_ref[...]
    hs = _block_diag_expand(DIMS, N_HEADS)
    he = _block_diag_expand(N_HEADS, DIMS)
    wlog = jnp.dot(q * k, hs, precision=_PREC)
    e = jnp.exp(wlog)
    ev = v * jnp.dot(e, he, precision=_PREC)
    pad = jnp.zeros((e.shape[0], YCOLS - DIMS - N_HEADS), jnp.float32)
    y_ref[...] = jnp.concatenate([ev, e, pad], axis=1)


def _ln_block(x, scale, bias):
    m = x.mean(axis=-1, keepdims=True)
    v = jnp.mean(jnp.square(x - m), axis=-1, keepdims=True)
    return (x - m) / jnp.sqrt(v + 1e-6) * scale + bias


def _pixel_kernel(acc_ref, tok_ref, ow_ref, ob_ref, ls_ref, lb_ref,
                  w1_ref, b1_ref, w2_ref, b2_ref, fs_ref, fb_ref,
                  qw_ref, qb_ref, x_ref, q_ref):
    s = acc_ref[0] + acc_ref[1]
    numer = s[:, :DIMS]
    norm = s[:, DIMS:DIMS + N_HEADS]
    he = _block_diag_expand(N_HEADS, DIMS)
    dinv = 1.0 / (norm + EPS)
    r = numer * jnp.dot(dinv, he, precision=_PREC)
    res = jnp.dot(r, ow_ref[...], precision=_PREC) + ob_ref[...] + tok_ref[...]
    h = _ln_block(res, ls_ref[...], lb_ref[...])
    f = jnp.dot(jax.nn.gelu(jnp.dot(h, w1_ref[...], precision=_PREC)
                            + b1_ref[...]),
                w2_ref[...], precision=_PREC) + b2_ref[...]
    x = _ln_block(h + f, fs_ref[...], fb_ref[...])
    x_ref[...] = x
    q_ref[...] = (jnp.dot(x, qw_ref[...], precision=_PREC)
                  + qb_ref[...]) * (1.0 / jnp.sqrt(jnp.float32(HEAD)))


def _mat_spec(shape):
    nd = len(shape)
    return pl.BlockSpec(shape, lambda i: (0,) * nd)


def _entry_l1_call(gemb, cnts2, ce, cw, cb, vw, vb):
    grid = (N_ENTRIES // BE,)
    return pl.pallas_call(
        _entry_l1_kernel,
        grid=grid,
        in_specs=[
            pl.BlockSpec((BE, DIMS), lambda i: (i, 0)),
            pl.BlockSpec((BE, 1), lambda i: (i, 0)),
            _mat_spec((len(COUNTS_BIN) + 1, DIMS)),
            _mat_spec((DIMS, DIMS)),
            _mat_spec((DIMS,)),
            _mat_spec((DIMS, DIMS)),
            _mat_spec((DIMS,)),
        ],
        out_specs=pl.BlockSpec((BE, YCOLS), lambda i: (i, 0)),
        out_shape=jax.ShapeDtypeStruct((N_ENTRIES, YCOLS), jnp.float32),
    )(gemb, cnts2, ce, cw, cb, vw, vb)


def _entry_call(gemb, cnts2, qexp, ce, cw, cb, kw, kb, vw, vb):
    grid = (N_ENTRIES // BE,)
    return pl.pallas_call(
        _entry_kernel,
        grid=grid,
        in_specs=[
            pl.BlockSpec((BE, DIMS), lambda i: (i, 0)),
            pl.BlockSpec((BE, 1), lambda i: (i, 0)),
            pl.BlockSpec((BE, DIMS), lambda i: (i, 0)),
            _mat_spec((len(COUNTS_BIN) + 1, DIMS)),
            _mat_spec((DIMS, DIMS)),
            _mat_spec((DIMS,)),
            _mat_spec((DIMS, DIMS)),
            _mat_spec((DIMS,)),
            _mat_spec((DIMS, DIMS)),
            _mat_spec((DIMS,)),
        ],
        out_specs=pl.BlockSpec((BE, YCOLS), lambda i: (i, 0)),
        out_shape=jax.ShapeDtypeStruct((N_ENTRIES, YCOLS), jnp.float32),
    )(gemb, cnts2, qexp, ce, cw, cb, kw, kb, vw, vb)


def _pixel_call(acc, tok, p, pn):
    grid = (N_PIXELS // BP,)
    return pl.pallas_call(
        _pixel_kernel,
        grid=grid,
        in_specs=[
            pl.BlockSpec((NC, BP, YCOLS), lambda i: (0, i, 0)),
            pl.BlockSpec((BP, DIMS), lambda i: (i, 0)),
            _mat_spec((DIMS, DIMS)),
            _mat_spec((DIMS,)),
            _mat_spec((DIMS,)),
            _mat_spec((DIMS,)),
            _mat_spec((DIMS, FFN_DIMS)),
            _mat_spec((FFN_DIMS,)),
            _mat_spec((FFN_DIMS, DIMS)),
            _mat_spec((DIMS,)),
            _mat_spec((DIMS,)),
            _mat_spec((DIMS,)),
            _mat_spec((DIMS, DIMS)),
            _mat_spec((DIMS,)),
        ],
        out_specs=[
            pl.BlockSpec((BP, DIMS), lambda i: (i, 0)),
            pl.BlockSpec((BP, DIMS), lambda i: (i, 0)),
        ],
        out_shape=[
            jax.ShapeDtypeStruct((N_PIXELS, DIMS), jnp.float32),
            jax.ShapeDtypeStruct((N_PIXELS, DIMS), jnp.float32),
        ],
    )(acc, tok, p['o_W'], p['o_b'], p['att_ln_s'], p['att_ln_b'],
      p['ffn_W1'], p['ffn_b1'], p['ffn_W2'], p['ffn_b2'],
      p['ffn_ln_s'], p['ffn_ln_b'], pn['q_W'], pn['q_b'])


def _gene_gather(table, idx):
    return _make_gather(N_ENTRIES, DIMS)(table, idx)


def _q_gather(table, idx):
    return _make_gather(N_ENTRIES, DIMS)(table, idx)


def _seg_reduce(y, segms, zrows):
    return _make_seg_reduce()(y, segms, zrows)


def kernel(indices, indptr, cnts, params):
    ind = jnp.zeros((N_ENTRIES,), jnp.int32).at[indptr[1:]].add(1, mode='drop')
    segms = jnp.cumsum(ind).astype(jnp.int32)
    cnts2 = cnts.reshape(N_ENTRIES, 1)
    zrows = jnp.zeros((N_PIXELS // NS, YCOLS), jnp.float32)
    tok0 = jnp.zeros((N_PIXELS, DIMS), jnp.float32)

    gemb = _gene_gather(params['gene_embed'], indices)

    ce, cw, cb = params['cnt_embed'], params['ctx_W'], params['ctx_b']
    layers = params['layers']

    y = _entry_l1_call(gemb, cnts2, ce, cw, cb,
                       layers[0]['v_W'], layers[0]['v_b'])
    acc = _seg_reduce(y, segms, zrows)
    x, q = _pixel_call(acc, tok0, layers[0], layers[1])

    for li in (1, 2):
        p = layers[li]
        qexp = _q_gather(q, segms)
        y = _entry_call(gemb, cnts2, qexp, ce, cw, cb,
                        p['k_W'], p['k_b'], p['v_W'], p['v_b'])
        acc = _seg_reduce(y, segms, zrows)
        pn = layers[li + 1] if li + 1 < N_LAYERS else p
        x, q = _pixel_call(acc, x, p, pn)

    return x.reshape(H, W, DIMS)
